# baseline (speedup 1.0000x reference)
import functools

import jax
import jax.numpy as jnp
from jax import lax
from jax.experimental import pallas as pl
from jax.experimental.pallas import tpu as pltpu
from jax.experimental.pallas import tpu_sc as plsc

_BATCH = 16384
_TABLE = 1000
_NC = 2
_NS = 16
_NW = _NC * _NS
_CHUNK = _BATCH // _NW
_L = 16
_VREGS = _CHUNK // _L


def _sqrt16(x):
    i = plsc.bitcast(x, jnp.int32)
    i = 0x5F3759DF - (i >> 1)
    y = plsc.bitcast(i, jnp.float32)
    xh = x * 0.5
    y = y * (1.5 - xh * y * y)
    y = y * (1.5 - xh * y * y)
    return x * y


@functools.partial(
    pl.kernel,
    mesh=plsc.VectorSubcoreMesh(core_axis_name="c", subcore_axis_name="s"),
    out_type=(
        jax.ShapeDtypeStruct((_BATCH,), jnp.float32),
        jax.ShapeDtypeStruct((_BATCH,), jnp.float32),
    ),
    scratch_types=[
        pltpu.VMEM((_CHUNK,), jnp.int32),
        pltpu.VMEM((_TABLE,), jnp.float32),
        pltpu.VMEM((_CHUNK,), jnp.float32),
        pltpu.VMEM((_CHUNK,), jnp.float32),
        pltpu.SemaphoreType.DMA,
        pltpu.SemaphoreType.DMA,
    ],
    compiler_params=pltpu.CompilerParams(
        needs_layout_passes=False,
        skip_device_barrier=True,
        disable_bounds_checks=True,
        disable_semaphore_checks=True,
    ),
)
def _encode_sc(t_hbm, tab_hbm, o0_hbm, o1_hbm, idx_v, tab_v, s0_v, s1_v,
               sem0, sem1):
    wid = lax.axis_index("s") * _NC + lax.axis_index("c")
    base = wid * _CHUNK
    cp_idx = pltpu.async_copy(t_hbm.at[pl.ds(base, _CHUNK)], idx_v, sem0)
    cp_tab = pltpu.async_copy(tab_hbm, tab_v, sem1)
    cp_idx.wait()
    cp_tab.wait()
    _UNROLL = 4

    def _step(i, _):
        for u in range(_UNROLL):
            j = i * _UNROLL + u
            sl = pl.ds(j * _L, _L)
            a = plsc.load_gather(tab_v, [idx_v[sl]])
            s0_v[sl] = _sqrt16(a)
            s1_v[sl] = _sqrt16(1.0 - a)
        return 0

    lax.fori_loop(0, _VREGS // _UNROLL, _step, 0)
    cp0 = pltpu.async_copy(s0_v, o0_hbm.at[pl.ds(base, _CHUNK)], sem0)
    cp1 = pltpu.async_copy(s1_v, o1_hbm.at[pl.ds(base, _CHUNK)], sem1)
    cp0.wait()
    cp1.wait()


def kernel(t, alphas_cumprod):
    s0, s1 = _encode_sc(t.astype(jnp.int32), alphas_cumprod)
    return jnp.concatenate([s0[:, None], s1[:, None]], axis=1)


# speedup vs baseline: 1.0094x; 1.0094x over previous
import functools

import jax
import jax.numpy as jnp
from jax import lax
from jax.experimental import pallas as pl
from jax.experimental.pallas import tpu as pltpu
from jax.experimental.pallas import tpu_sc as plsc

_BATCH = 16384
_TABLE = 1000
_NC = 2
_NS = 16
_NW = _NC * _NS
_CHUNK = _BATCH // _NW
_L = 16
_VREGS = _CHUNK // _L


def _sqrt16(x):
    i = plsc.bitcast(x, jnp.int32)
    i = 0x5F3759DF - (i >> 1)
    y = plsc.bitcast(i, jnp.float32)
    xh = x * 0.5
    y = y * (1.5 - xh * y * y)
    y = y * (1.5 - xh * y * y)
    return x * y


@functools.partial(
    pl.kernel,
    mesh=plsc.VectorSubcoreMesh(core_axis_name="c", subcore_axis_name="s"),
    out_type=(
        jax.ShapeDtypeStruct((_BATCH,), jnp.float32),
        jax.ShapeDtypeStruct((_BATCH,), jnp.float32),
    ),
    scratch_types=[
        pltpu.VMEM((_CHUNK,), jnp.int32),
        pltpu.VMEM((_TABLE,), jnp.float32),
        pltpu.VMEM((_CHUNK,), jnp.float32),
        pltpu.VMEM((_CHUNK,), jnp.float32),
        pltpu.SemaphoreType.DMA,
        pltpu.SemaphoreType.DMA,
    ],
    compiler_params=pltpu.CompilerParams(
        needs_layout_passes=False,
        skip_device_barrier=True,
        disable_bounds_checks=True,
        disable_semaphore_checks=True,
    ),
)
def _encode_sc(t_hbm, tab_hbm, o0_hbm, o1_hbm, idx_v, tab_v, s0_v, s1_v,
               sem0, sem1):
    wid = lax.axis_index("s") * _NC + lax.axis_index("c")
    base = wid * _CHUNK
    cp_idx = pltpu.async_copy(t_hbm.at[pl.ds(base, _CHUNK)], idx_v, sem0)
    cp_tab = pltpu.async_copy(tab_hbm, tab_v, sem1)
    cp_idx.wait()
    cp_tab.wait()
    _UNROLL = 2

    def _step(i, _):
        for u in range(_UNROLL):
            j = i * _UNROLL + u
            sl = pl.ds(j * _L, _L)
            a = plsc.load_gather(tab_v, [idx_v[sl]])
            s0_v[sl] = _sqrt16(a)
            s1_v[sl] = _sqrt16(1.0 - a)
        return 0

    lax.fori_loop(0, _VREGS // _UNROLL, _step, 0)
    cp0 = pltpu.async_copy(s0_v, o0_hbm.at[pl.ds(base, _CHUNK)], sem0)
    cp1 = pltpu.async_copy(s1_v, o1_hbm.at[pl.ds(base, _CHUNK)], sem1)
    cp0.wait()
    cp1.wait()


def kernel(t, alphas_cumprod):
    s0, s1 = _encode_sc(t.astype(jnp.int32), alphas_cumprod)
    return jnp.stack([s0, s1], axis=-1)


# speedup vs baseline: 1.0178x; 1.0083x over previous
import functools

import jax
import jax.numpy as jnp
from jax import lax
from jax.experimental import pallas as pl
from jax.experimental.pallas import tpu as pltpu
from jax.experimental.pallas import tpu_sc as plsc

_BATCH = 16384
_TABLE = 1000
_NC = 2
_NS = 16
_NW = _NC * _NS
_CHUNK = _BATCH // _NW
_L = 16
_VREGS = _CHUNK // _L


def _sqrt16(x):
    i = plsc.bitcast(x, jnp.int32)
    i = 0x5F3759DF - (i >> 1)
    y = plsc.bitcast(i, jnp.float32)
    xh = x * 0.5
    y = y * (1.5 - xh * y * y)
    y = y * (1.5 - xh * y * y)
    return x * y


@functools.partial(
    pl.kernel,
    mesh=plsc.VectorSubcoreMesh(core_axis_name="c", subcore_axis_name="s"),
    out_type=(
        jax.ShapeDtypeStruct((_BATCH,), jnp.float32),
        jax.ShapeDtypeStruct((_BATCH,), jnp.float32),
    ),
    scratch_types=[
        pltpu.VMEM((_CHUNK,), jnp.int32),
        pltpu.VMEM((_TABLE,), jnp.float32),
        pltpu.VMEM((_CHUNK,), jnp.float32),
        pltpu.VMEM((_CHUNK,), jnp.float32),
        pltpu.SemaphoreType.DMA,
        pltpu.SemaphoreType.DMA,
    ],
    compiler_params=pltpu.CompilerParams(
        needs_layout_passes=False,
        skip_device_barrier=True,
        disable_bounds_checks=True,
        disable_semaphore_checks=True,
    ),
)
def _encode_sc(t_hbm, tab_hbm, o0_hbm, o1_hbm, idx_v, tab_v, s0_v, s1_v,
               sem0, sem1):
    wid = lax.axis_index("s") * _NC + lax.axis_index("c")
    base = wid * _CHUNK
    cp_idx = pltpu.async_copy(t_hbm.at[pl.ds(base, _CHUNK)], idx_v, sem0)
    cp_tab = pltpu.async_copy(tab_hbm, tab_v, sem1)
    cp_idx.wait()
    cp_tab.wait()
    _UNROLL = 1

    def _step(i, _):
        for u in range(_UNROLL):
            j = i * _UNROLL + u
            sl = pl.ds(j * _L, _L)
            a = plsc.load_gather(tab_v, [idx_v[sl]])
            s0_v[sl] = _sqrt16(a)
            s1_v[sl] = _sqrt16(1.0 - a)
        return 0

    lax.fori_loop(0, _VREGS // _UNROLL, _step, 0)
    cp0 = pltpu.async_copy(s0_v, o0_hbm.at[pl.ds(base, _CHUNK)], sem0)
    cp1 = pltpu.async_copy(s1_v, o1_hbm.at[pl.ds(base, _CHUNK)], sem1)
    cp0.wait()
    cp1.wait()


def kernel(t, alphas_cumprod):
    s0, s1 = _encode_sc(t.astype(jnp.int32), alphas_cumprod)
    return jnp.stack([s0, s1], axis=-1)


# speedup vs baseline: 1.0190x; 1.0012x over previous
import functools

import jax
import jax.numpy as jnp
from jax import lax
from jax.experimental import pallas as pl
from jax.experimental.pallas import tpu as pltpu
from jax.experimental.pallas import tpu_sc as plsc

_BATCH = 16384
_TABLE = 1000
_NC = 2
_NS = 16
_NW = _NC * _NS
_CHUNK = _BATCH // _NW
_L = 16
_VREGS = _CHUNK // _L


def _sqrt16(x):
    i = plsc.bitcast(x, jnp.int32)
    i = 0x5F3759DF - (i >> 1)
    y = plsc.bitcast(i, jnp.float32)
    xh = x * 0.5
    y = y * (1.5 - xh * y * y)
    y = y * (1.5 - xh * y * y)
    return x * y


@functools.partial(
    pl.kernel,
    mesh=plsc.VectorSubcoreMesh(core_axis_name="c", subcore_axis_name="s"),
    out_type=(
        jax.ShapeDtypeStruct((_BATCH,), jnp.float32),
        jax.ShapeDtypeStruct((_BATCH,), jnp.float32),
    ),
    scratch_types=[
        pltpu.VMEM((_CHUNK,), jnp.int32),
        pltpu.VMEM((_TABLE,), jnp.float32),
        pltpu.VMEM((_CHUNK,), jnp.float32),
        pltpu.VMEM((_CHUNK,), jnp.float32),
        pltpu.SemaphoreType.DMA,
        pltpu.SemaphoreType.DMA,
    ],
    compiler_params=pltpu.CompilerParams(
        needs_layout_passes=False,
        skip_device_barrier=True,
        disable_bounds_checks=True,
        disable_semaphore_checks=True,
    ),
)
def _encode_sc(t_hbm, tab_hbm, o0_hbm, o1_hbm, idx_v, tab_v, s0_v, s1_v,
               sem0, sem1):
    wid = lax.axis_index("s") * _NC + lax.axis_index("c")
    base = wid * _CHUNK
    cp_idx = pltpu.async_copy(t_hbm.at[pl.ds(base, _CHUNK)], idx_v, sem0)
    cp_tab = pltpu.async_copy(tab_hbm, tab_v, sem1)
    cp_idx.wait()
    cp_tab.wait()
    @plsc.parallel_loop(0, _CHUNK, _L, unroll=2)
    def _step(i):
        sl = pl.ds(i, _L)
        a = plsc.load_gather(tab_v, [idx_v[sl]])
        s0_v[sl] = _sqrt16(a)
        s1_v[sl] = _sqrt16(1.0 - a)
    cp0 = pltpu.async_copy(s0_v, o0_hbm.at[pl.ds(base, _CHUNK)], sem0)
    cp1 = pltpu.async_copy(s1_v, o1_hbm.at[pl.ds(base, _CHUNK)], sem1)
    cp0.wait()
    cp1.wait()


def kernel(t, alphas_cumprod):
    s0, s1 = _encode_sc(t.astype(jnp.int32), alphas_cumprod)
    return jnp.stack([s0, s1], axis=-1)


# speedup vs baseline: 1.0226x; 1.0035x over previous
import functools

import jax
import jax.numpy as jnp
from jax import lax
from jax.experimental import pallas as pl
from jax.experimental.pallas import tpu as pltpu
from jax.experimental.pallas import tpu_sc as plsc

_BATCH = 16384
_TABLE = 1000
_NC = 2
_NS = 16
_NW = _NC * _NS
_CHUNK = _BATCH // _NW
_L = 16


def _sqrt16(x):
    i = plsc.bitcast(x, jnp.int32)
    i = 0x5F3759DF - (i >> 1)
    y = plsc.bitcast(i, jnp.float32)
    xh = x * 0.5
    y = y * (1.5 - xh * y * y)
    y = y * (1.5 - xh * y * y)
    return x * y


@functools.partial(
    pl.kernel,
    mesh=plsc.VectorSubcoreMesh(core_axis_name="c", subcore_axis_name="s"),
    out_type=(
        jax.ShapeDtypeStruct((_BATCH,), jnp.float32),
        jax.ShapeDtypeStruct((_BATCH,), jnp.float32),
    ),
    scratch_types=[
        pltpu.VMEM((_CHUNK,), jnp.int32),
        pltpu.VMEM((_TABLE,), jnp.float32),
        pltpu.VMEM((_CHUNK,), jnp.float32),
        pltpu.VMEM((_CHUNK,), jnp.float32),
        pltpu.SemaphoreType.DMA,
        pltpu.SemaphoreType.DMA,
    ],
    compiler_params=pltpu.CompilerParams(
        needs_layout_passes=False,
        skip_device_barrier=True,
        disable_bounds_checks=True,
        disable_semaphore_checks=True,
    ),
)
def _encode_sc(t_hbm, tab_hbm, o0_hbm, o1_hbm, idx_v, tab_v, s0_v, s1_v,
               sem0, sem1):
    wid = lax.axis_index("s") * _NC + lax.axis_index("c")
    base = wid * _CHUNK
    cp_idx = pltpu.async_copy(t_hbm.at[pl.ds(base, _CHUNK)], idx_v, sem0)
    cp_tab = pltpu.async_copy(tab_hbm, tab_v, sem1)
    cp_idx.wait()
    cp_tab.wait()
    @plsc.parallel_loop(0, _CHUNK, _L, unroll=2)
    def _step(i):
        sl = pl.ds(i, _L)
        a = plsc.load_gather(tab_v, [idx_v[sl]])
        s0_v[sl] = _sqrt16(a)
        s1_v[sl] = _sqrt16(1.0 - a)
    cp0 = pltpu.async_copy(s0_v, o0_hbm.at[pl.ds(base, _CHUNK)], sem0)
    cp1 = pltpu.async_copy(s1_v, o1_hbm.at[pl.ds(base, _CHUNK)], sem1)
    cp0.wait()
    cp1.wait()


def kernel(t, alphas_cumprod):
    s0, s1 = _encode_sc(t.astype(jnp.int32), alphas_cumprod)
    return jnp.stack([s0, s1], axis=-1)
